# baseline (speedup 1.0000x reference)
import functools
import jax
import jax.numpy as jnp
from jax import lax
from jax.experimental import pallas as pl
from jax.experimental.pallas import tpu as pltpu
from jax.experimental.pallas import tpu_sc as plsc

BATCH = 4
SEQ = 2048
DM = 2048
NC = 2
NS = 16
NW = NC * NS
ROWS_PER_W = SEQ // NW
CHUNK = 16
N_CHUNKS = ROWS_PER_W // CHUNK


def _sc_body(x_hbm, p_hbm, o_hbm, xbuf, pbuf):
    wid = lax.axis_index("c") * NS + lax.axis_index("s")
    row_base = wid * ROWS_PER_W

    def chunk_body(c, carry):
        row0 = row_base + c * CHUNK
        pltpu.sync_copy(p_hbm.at[pl.ds(row0, CHUNK), :], pbuf)

        def batch_body(b, carry2):
            pltpu.sync_copy(x_hbm.at[b, pl.ds(row0, CHUNK), :], xbuf)
            for r in range(CHUNK):
                def vbody(j, r=r):
                    xbuf[r, pl.ds(j, 16)] = (
                        xbuf[r, pl.ds(j, 16)] + pbuf[r, pl.ds(j, 16)]
                    )
                plsc.parallel_loop(0, DM, step=16, unroll=8)(vbody)
            pltpu.sync_copy(xbuf, o_hbm.at[b, pl.ds(row0, CHUNK), :])
            return carry2

        return lax.fori_loop(0, BATCH, batch_body, carry)

    lax.fori_loop(0, N_CHUNKS, chunk_body, 0)


def kernel(inputs, pos_table):
    mesh = plsc.VectorSubcoreMesh(core_axis_name="c", subcore_axis_name="s")
    k = functools.partial(
        pl.kernel,
        mesh=mesh,
        out_type=jax.ShapeDtypeStruct((BATCH, SEQ, DM), jnp.float32),
        scratch_types=[
            pltpu.VMEM((CHUNK, DM), jnp.float32),
            pltpu.VMEM((CHUNK, DM), jnp.float32),
        ],
    )(_sc_body)
    return k(inputs, pos_table)


# speedup vs baseline: 1.3352x; 1.3352x over previous
import functools
import jax
import jax.numpy as jnp
from jax import lax
from jax.experimental import pallas as pl
from jax.experimental.pallas import tpu as pltpu
from jax.experimental.pallas import tpu_sc as plsc

BATCH = 4
SEQ = 2048
DM = 2048
NC = 2
NS = 16
NW = NC * NS
ROWS_PER_W = SEQ // NW
CHUNK = 8
N_CHUNKS = ROWS_PER_W // CHUNK
VECS = CHUNK * DM // 16


def _sc_body(x_hbm, p_hbm, o_hbm,
             xbuf0, xbuf1, pbuf0, pbuf1,
             sx0, sx1, sp0, sp1, so0, so1):
    wid = lax.axis_index("c") * NS + lax.axis_index("s")
    row_base = wid * ROWS_PER_W
    xbufs, pbufs = (xbuf0, xbuf1), (pbuf0, pbuf1)
    sxs, sps, sos = (sx0, sx1), (sp0, sp1), (so0, so1)

    def x_src(c, b):
        return x_hbm.at[b, pl.ds(row_base + c * CHUNK, CHUNK), :]

    def o_dst(c, b):
        return o_hbm.at[b, pl.ds(row_base + c * CHUNK, CHUNK), :]

    def p_src(c):
        return p_hbm.at[pl.ds(row_base + c * CHUNK, CHUNK), :]

    items = [(c, b) for c in range(N_CHUNKS) for b in range(BATCH)]
    n = len(items)

    pltpu.make_async_copy(p_src(0), pbuf0, sp0).start()
    pltpu.make_async_copy(x_src(0, 0), xbuf0, sx0).start()

    for i, (c, b) in enumerate(items):
        s = i % 2
        ps = c % 2
        if b == 0:
            pltpu.make_async_copy(p_src(c), pbufs[ps], sps[ps]).wait()
            if c + 1 < N_CHUNKS:
                nps = (c + 1) % 2
                pltpu.make_async_copy(p_src(c + 1), pbufs[nps], sps[nps]).start()
        if i + 1 < n:
            ns = (i + 1) % 2
            if i >= 1:
                pc, pb = items[i - 1]
                pltpu.make_async_copy(xbufs[ns], o_dst(pc, pb), sos[ns]).wait()
            nc, nb = items[i + 1]
            pltpu.make_async_copy(x_src(nc, nb), xbufs[ns], sxs[ns]).start()
        pltpu.make_async_copy(x_src(c, b), xbufs[s], sxs[s]).wait()

        xb, pb_ = xbufs[s], pbufs[ps]

        def vbody(j, xb=xb, pb_=pb_):
            for r in range(CHUNK):
                xb[r, pl.ds(j, 16)] = xb[r, pl.ds(j, 16)] + pb_[r, pl.ds(j, 16)]

        plsc.parallel_loop(0, DM, step=16, unroll=2)(vbody)

        pltpu.make_async_copy(xbufs[s], o_dst(c, b), sos[s]).start()

    c0, b0 = items[n - 2]
    c1, b1 = items[n - 1]
    pltpu.make_async_copy(xbufs[(n - 2) % 2], o_dst(c0, b0), sos[(n - 2) % 2]).wait()
    pltpu.make_async_copy(xbufs[(n - 1) % 2], o_dst(c1, b1), sos[(n - 1) % 2]).wait()


def kernel(inputs, pos_table):
    mesh = plsc.VectorSubcoreMesh(core_axis_name="c", subcore_axis_name="s")
    k = functools.partial(
        pl.kernel,
        mesh=mesh,
        out_type=jax.ShapeDtypeStruct((BATCH, SEQ, DM), jnp.float32),
        scratch_types=[
            pltpu.VMEM((CHUNK, DM), jnp.float32),
            pltpu.VMEM((CHUNK, DM), jnp.float32),
            pltpu.VMEM((CHUNK, DM), jnp.float32),
            pltpu.VMEM((CHUNK, DM), jnp.float32),
            pltpu.SemaphoreType.DMA,
            pltpu.SemaphoreType.DMA,
            pltpu.SemaphoreType.DMA,
            pltpu.SemaphoreType.DMA,
            pltpu.SemaphoreType.DMA,
            pltpu.SemaphoreType.DMA,
        ],
    )(_sc_body)
    return k(inputs, pos_table)


# speedup vs baseline: 1.4268x; 1.0686x over previous
import functools
import jax
import jax.numpy as jnp
from jax import lax
from jax.experimental import pallas as pl
from jax.experimental.pallas import tpu as pltpu
from jax.experimental.pallas import tpu_sc as plsc

BATCH = 4
SEQ = 2048
DM = 2048
NC = 2
NS = 16
NW = NC * NS
ROWS_PER_W = SEQ // NW
CHUNK = 8
N_CHUNKS = ROWS_PER_W // CHUNK
NB = 3


def _sc_body(x_hbm, p_hbm, o_hbm,
             xbuf0, xbuf1, xbuf2, pbuf0, pbuf1,
             sx0, sx1, sx2, sp0, sp1, so0, so1, so2):
    wid = lax.axis_index("c") * NS + lax.axis_index("s")
    row_base = wid * ROWS_PER_W
    xbufs = (xbuf0, xbuf1, xbuf2)
    pbufs = (pbuf0, pbuf1)
    sxs = (sx0, sx1, sx2)
    sps = (sp0, sp1)
    sos = (so0, so1, so2)

    def x_src(c, b):
        return x_hbm.at[b, pl.ds(row_base + c * CHUNK, CHUNK), :]

    def o_dst(c, b):
        return o_hbm.at[b, pl.ds(row_base + c * CHUNK, CHUNK), :]

    def p_src(c):
        return p_hbm.at[pl.ds(row_base + c * CHUNK, CHUNK), :]

    items = [(c, b) for c in range(N_CHUNKS) for b in range(BATCH)]
    n = len(items)

    pltpu.make_async_copy(p_src(0), pbuf0, sp0).start()
    pltpu.make_async_copy(x_src(0, 0), xbuf0, sx0).start()

    for i, (c, b) in enumerate(items):
        s = i % NB
        ps = c % 2
        if b == 0:
            pltpu.make_async_copy(p_src(c), pbufs[ps], sps[ps]).wait()
            if c + 1 < N_CHUNKS:
                nps = (c + 1) % 2
                pltpu.make_async_copy(p_src(c + 1), pbufs[nps], sps[nps]).start()
        if i + 1 < n:
            ns = (i + 1) % NB
            if i >= NB - 1:
                pc, pb = items[i + 1 - NB]
                pltpu.make_async_copy(xbufs[ns], o_dst(pc, pb), sos[ns]).wait()
            nc, nb = items[i + 1]
            pltpu.make_async_copy(x_src(nc, nb), xbufs[ns], sxs[ns]).start()
        pltpu.make_async_copy(x_src(c, b), xbufs[s], sxs[s]).wait()

        xb, pb_ = xbufs[s], pbufs[ps]

        def vbody(j, xb=xb, pb_=pb_):
            for r in range(CHUNK):
                xb[r, pl.ds(j, 16)] = xb[r, pl.ds(j, 16)] + pb_[r, pl.ds(j, 16)]

        plsc.parallel_loop(0, DM, step=16, unroll=2)(vbody)

        pltpu.make_async_copy(xbufs[s], o_dst(c, b), sos[s]).start()

    for i in range(n - NB, n):
        ce, be = items[i]
        pltpu.make_async_copy(xbufs[i % NB], o_dst(ce, be), sos[i % NB]).wait()


def kernel(inputs, pos_table):
    mesh = plsc.VectorSubcoreMesh(core_axis_name="c", subcore_axis_name="s")
    k = functools.partial(
        pl.kernel,
        mesh=mesh,
        out_type=jax.ShapeDtypeStruct((BATCH, SEQ, DM), jnp.float32),
        scratch_types=[
            pltpu.VMEM((CHUNK, DM), jnp.float32),
            pltpu.VMEM((CHUNK, DM), jnp.float32),
            pltpu.VMEM((CHUNK, DM), jnp.float32),
            pltpu.VMEM((CHUNK, DM), jnp.float32),
            pltpu.VMEM((CHUNK, DM), jnp.float32),
            pltpu.SemaphoreType.DMA,
            pltpu.SemaphoreType.DMA,
            pltpu.SemaphoreType.DMA,
            pltpu.SemaphoreType.DMA,
            pltpu.SemaphoreType.DMA,
            pltpu.SemaphoreType.DMA,
            pltpu.SemaphoreType.DMA,
            pltpu.SemaphoreType.DMA,
        ],
    )(_sc_body)
    return k(inputs, pos_table)


# speedup vs baseline: 1.4586x; 1.0223x over previous
import functools
import jax
import jax.numpy as jnp
from jax import lax
from jax.experimental import pallas as pl
from jax.experimental.pallas import tpu as pltpu
from jax.experimental.pallas import tpu_sc as plsc

BATCH = 4
SEQ = 2048
DM = 2048
NC = 2
NS = 16
NW = NC * NS
ROWS_PER_W = SEQ // NW
CHUNK = 8
N_CHUNKS = ROWS_PER_W // CHUNK
NB = 3


def _sc_body(x_hbm, p_hbm, o_hbm,
             xbuf0, xbuf1, xbuf2, pbuf0, pbuf1,
             sx0, sx1, sx2, sp0, sp1, so0, so1, so2):
    wid = lax.axis_index("c") * NS + lax.axis_index("s")
    row_base = wid * ROWS_PER_W
    xbufs = (xbuf0, xbuf1, xbuf2)
    pbufs = (pbuf0, pbuf1)
    sxs = (sx0, sx1, sx2)
    sps = (sp0, sp1)
    sos = (so0, so1, so2)

    def x_src(c, b):
        return x_hbm.at[b, pl.ds(row_base + c * CHUNK, CHUNK), :]

    def o_dst(c, b):
        return o_hbm.at[b, pl.ds(row_base + c * CHUNK, CHUNK), :]

    def p_src(c):
        return p_hbm.at[pl.ds(row_base + c * CHUNK, CHUNK), :]

    items = [(c, b) for c in range(N_CHUNKS) for b in range(BATCH)]
    n = len(items)

    pltpu.make_async_copy(p_src(0), pbuf0, sp0).start()
    pltpu.make_async_copy(x_src(0, 0), xbuf0, sx0).start()

    for i, (c, b) in enumerate(items):
        s = i % NB
        ps = c % 2
        if b == 0:
            pltpu.make_async_copy(p_src(c), pbufs[ps], sps[ps]).wait()
            if c + 1 < N_CHUNKS:
                nps = (c + 1) % 2
                pltpu.make_async_copy(p_src(c + 1), pbufs[nps], sps[nps]).start()
        if i + 1 < n:
            ns = (i + 1) % NB
            if i >= NB - 1:
                pc, pb = items[i + 1 - NB]
                pltpu.make_async_copy(xbufs[ns], o_dst(pc, pb), sos[ns]).wait()
            nc, nb = items[i + 1]
            pltpu.make_async_copy(x_src(nc, nb), xbufs[ns], sxs[ns]).start()
        pltpu.make_async_copy(x_src(c, b), xbufs[s], sxs[s]).wait()

        xb, pb_ = xbufs[s], pbufs[ps]

        def vbody(j, xb=xb, pb_=pb_):
            for r in range(CHUNK):
                plsc.addupdate(xb.at[r, pl.ds(j, 16)], pb_[r, pl.ds(j, 16)])

        plsc.parallel_loop(0, DM, step=16, unroll=2)(vbody)

        pltpu.make_async_copy(xbufs[s], o_dst(c, b), sos[s]).start()

    for i in range(n - NB, n):
        ce, be = items[i]
        pltpu.make_async_copy(xbufs[i % NB], o_dst(ce, be), sos[i % NB]).wait()


def kernel(inputs, pos_table):
    mesh = plsc.VectorSubcoreMesh(core_axis_name="c", subcore_axis_name="s")
    k = functools.partial(
        pl.kernel,
        mesh=mesh,
        out_type=jax.ShapeDtypeStruct((BATCH, SEQ, DM), jnp.float32),
        scratch_types=[
            pltpu.VMEM((CHUNK, DM), jnp.float32),
            pltpu.VMEM((CHUNK, DM), jnp.float32),
            pltpu.VMEM((CHUNK, DM), jnp.float32),
            pltpu.VMEM((CHUNK, DM), jnp.float32),
            pltpu.VMEM((CHUNK, DM), jnp.float32),
            pltpu.SemaphoreType.DMA,
            pltpu.SemaphoreType.DMA,
            pltpu.SemaphoreType.DMA,
            pltpu.SemaphoreType.DMA,
            pltpu.SemaphoreType.DMA,
            pltpu.SemaphoreType.DMA,
            pltpu.SemaphoreType.DMA,
            pltpu.SemaphoreType.DMA,
        ],
    )(_sc_body)
    return k(inputs, pos_table)
